# baseline (speedup 1.0000x reference)
import functools

import jax
import jax.numpy as jnp
from jax import lax
from jax.experimental import pallas as pl
from jax.experimental.pallas import tpu as pltpu
from jax.experimental.pallas import tpu_sc as plsc

N_NODES = 10000
N_EDGES = 320000
D = 128
DH = D // 2

NC = 2
NS = 16
CB = 128
NBUF = 4
KA = 2
G = 16
K = G * (-(-N_EDGES // (NS * CB * G)))
NG = K // G
E_PAD = NS * K * CB
N_PAD = 10240
RPT = N_PAD // NS


def _sc_aggregate(x2cat, src5, dst4, zeros):
    mesh = plsc.VectorSubcoreMesh(core_axis_name="c", subcore_axis_name="s")

    @functools.partial(
        pl.kernel,
        out_type=(
            jax.ShapeDtypeStruct((NC, N_PAD, DH), jnp.float32),
            jax.ShapeDtypeStruct((NC, N_PAD), jnp.float32),
        ),
        mesh=mesh,
        compiler_params=pltpu.CompilerParams(use_tc_tiling_on_sc=False,
                                             needs_layout_passes=False),
        scratch_types=[
            pltpu.VMEM((G, CB), jnp.int32),
            pltpu.VMEM((G, CB), jnp.int32),
            pltpu.VMEM((NBUF, CB, DH), jnp.float32),
            pltpu.VMEM((N_PAD,), jnp.float32),
            pltpu.VMEM((RPT,), jnp.float32),
            pltpu.VMEM((RPT,), jnp.float32),
            pltpu.VMEM_SHARED((N_PAD, DH), jnp.float32),
            pltpu.VMEM_SHARED((NS, N_PAD), jnp.float32),
            pltpu.SemaphoreType.DMA((NBUF,)),
            pltpu.SemaphoreType.DMA((NBUF,)),
        ],
    )
    def agg_kernel(x_hbm, src_hbm, dst_hbm, z_hbm, agg_out, deg_out,
                   src_v, dst_v, rows_v, dloc_v, dacc_v, dtmp_v,
                   agg_s, dslab_s, gsem, ssem):
        c = lax.axis_index("c")
        s = lax.axis_index("s")
        row0 = s * RPT

        pltpu.sync_copy(z_hbm, agg_s.at[pl.ds(row0, RPT)])

        def zinit(k, carry):
            dloc_v[pl.ds(k * 16, 16)] = jnp.zeros((16,), jnp.float32)
            return carry

        lax.fori_loop(0, N_PAD // 16, zinit, 0)

        ones16 = jnp.ones((16,), jnp.float32)

        plsc.subcore_barrier()

        def group(g, carry):
            base = g * G
            pltpu.sync_copy(src_hbm.at[c, s, pl.ds(base, G)], src_v)
            pltpu.sync_copy(dst_hbm.at[s, pl.ds(base, G)], dst_v)
            for b in range(NBUF - KA):
                pltpu.async_copy(x_hbm.at[src_v.at[b]], rows_v.at[b],
                                 gsem.at[b])
            for i in range(G):
                b = i % NBUF
                pltpu.make_async_copy(
                    x_hbm.at[src_v.at[i]], rows_v.at[b], gsem.at[b]).wait()
                pltpu.async_copy(rows_v.at[b], agg_s.at[dst_v.at[i]],
                                 ssem.at[b], add=True)
                for r in range(CB // 16):
                    dv = dst_v[i, pl.ds(r * 16, 16)]
                    plsc.addupdate_scatter(dloc_v, [dv], ones16)
                ni = i + NBUF - KA
                if ni < G:
                    nb = ni % NBUF
                    if i >= KA:
                        pltpu.make_async_copy(
                            rows_v.at[nb], agg_s.at[dst_v.at[i - KA]],
                            ssem.at[nb]).wait()
                    pltpu.async_copy(x_hbm.at[src_v.at[ni]], rows_v.at[nb],
                                     gsem.at[nb])
            for i in range(G - NBUF, G):
                pltpu.make_async_copy(
                    rows_v.at[i % NBUF], agg_s.at[dst_v.at[i]],
                    ssem.at[i % NBUF]).wait()
            return carry

        lax.fori_loop(0, NG, group, 0)

        pltpu.sync_copy(dloc_v, dslab_s.at[s])

        plsc.subcore_barrier()

        pltpu.sync_copy(dslab_s.at[0, pl.ds(row0, RPT)], dacc_v)
        for t in range(1, NS):
            pltpu.sync_copy(dslab_s.at[t, pl.ds(row0, RPT)], dtmp_v)

            def addv(k, carry):
                sl = pl.ds(k * 16, 16)
                dacc_v[sl] = dacc_v[sl] + dtmp_v[sl]
                return carry

            lax.fori_loop(0, RPT // 16, addv, 0)

        pltpu.sync_copy(agg_s.at[pl.ds(row0, RPT)],
                        agg_out.at[c, pl.ds(row0, RPT)])
        pltpu.sync_copy(dacc_v, deg_out.at[c, pl.ds(row0, RPT)])

    return agg_kernel(x2cat, src5, dst4, zeros)


def _tc_update(agg2, deg2, x_pad, W, B):
    BR = 512
    grid = (N_PAD // BR,)

    def body(agg_ref, deg_ref, x_ref, w_ref, b_ref, o_ref):
        agg = jnp.concatenate([agg_ref[0], agg_ref[1]], axis=1)
        dg = (deg_ref[0] + deg_ref[1]) * 0.5
        dg = jnp.where(dg == 0.0, 1.0, dg)
        aggn = agg / dg[:, None]
        mm1 = lax.dot_general(aggn, w_ref[...], (((1,), (1,)), ((), ())),
                              preferred_element_type=jnp.float32)
        mm2 = lax.dot_general(x_ref[...], b_ref[...], (((1,), (1,)), ((), ())),
                              preferred_element_type=jnp.float32)
        o_ref[...] = mm1 + mm2

    return pl.pallas_call(
        body,
        grid=grid,
        in_specs=[
            pl.BlockSpec((NC, BR, DH), lambda i: (0, i, 0)),
            pl.BlockSpec((NC, BR), lambda i: (0, i)),
            pl.BlockSpec((BR, D), lambda i: (i, 0)),
            pl.BlockSpec((D, D), lambda i: (0, 0)),
            pl.BlockSpec((D, D), lambda i: (0, 0)),
        ],
        out_specs=pl.BlockSpec((BR, D), lambda i: (i, 0)),
        out_shape=jax.ShapeDtypeStruct((N_PAD, D), jnp.float32),
    )(agg2, deg2, x_pad, W, B)


def kernel(x, edge_index, W, B):
    src = edge_index[0].astype(jnp.int32)
    dst = edge_index[1].astype(jnp.int32)
    pad = E_PAD - N_EDGES
    src = jnp.concatenate([src, jnp.zeros((pad,), jnp.int32)])
    dst = jnp.concatenate([dst, jnp.full((pad,), N_NODES, jnp.int32)])
    src4 = src.reshape(NS, K, CB)
    dst4 = dst.reshape(NS, K, CB)
    src5 = jnp.stack([src4, src4 + N_PAD])

    x_pad = jnp.zeros((N_PAD, D), jnp.float32).at[:N_NODES].set(x)
    x2cat = x_pad.reshape(N_PAD, 2, DH).swapaxes(0, 1).reshape(2 * N_PAD, DH)
    zeros = jnp.zeros((RPT, DH), jnp.float32)

    agg2, deg2 = _sc_aggregate(x2cat, src5, dst4, zeros)
    out = _tc_update(agg2, deg2, x_pad, W, B)
    return out[:N_NODES]


# speedup vs baseline: 1.7436x; 1.7436x over previous
import functools

import jax
import jax.numpy as jnp
from jax import lax
from jax.experimental import pallas as pl
from jax.experimental.pallas import tpu as pltpu
from jax.experimental.pallas import tpu_sc as plsc

N_NODES = 10000
N_EDGES = 320000
D = 128
DH = D // 2

NC = 2
NS = 16
CB = 128
NBUF = 3
KA = 1
G = 16
K = G * (-(-N_EDGES // (NS * CB * G)))
NG = K // G
E_PAD = NS * K * CB
N_PAD = 10240
RPT = N_PAD // NS


def _sc_aggregate(x2cat, src4, dst4, zeros):
    mesh = plsc.VectorSubcoreMesh(core_axis_name="c", subcore_axis_name="s")

    @functools.partial(
        pl.kernel,
        out_type=(
            jax.ShapeDtypeStruct((NC, N_PAD, DH), jnp.float32),
            jax.ShapeDtypeStruct((NC, NS, N_PAD), jnp.float32),
        ),
        mesh=mesh,
        compiler_params=pltpu.CompilerParams(use_tc_tiling_on_sc=False,
                                             needs_layout_passes=False),
        scratch_types=[
            pltpu.VMEM((G, CB), jnp.int32),
            pltpu.VMEM((G, CB), jnp.int32),
            pltpu.VMEM((NBUF, CB, DH), jnp.float32),
            pltpu.VMEM((N_PAD,), jnp.float32),
            pltpu.VMEM_SHARED((N_PAD, DH), jnp.float32),
            pltpu.VMEM_SHARED((N_PAD, DH), jnp.float32),
            pltpu.SemaphoreType.DMA((NBUF,)),
            pltpu.SemaphoreType.DMA((NBUF,)),
        ],
    )
    def agg_kernel(x_hbm, src_hbm, dst_hbm, z_hbm, agg_out, deg_out,
                   src_v, dst_v, rows_v, dloc_v, xc_s, agg_s, gsem, ssem):
        c = lax.axis_index("c")
        s = lax.axis_index("s")
        row0 = s * RPT

        pltpu.sync_copy(x_hbm.at[pl.ds(c * N_PAD + row0, RPT)],
                        xc_s.at[pl.ds(row0, RPT)])
        pltpu.sync_copy(z_hbm, agg_s.at[pl.ds(row0, RPT)])

        def zinit(k, carry):
            dloc_v[pl.ds(k * 16, 16)] = jnp.zeros((16,), jnp.float32)
            return carry

        lax.fori_loop(0, N_PAD // 16, zinit, 0)

        ones16 = jnp.ones((16,), jnp.float32)

        plsc.subcore_barrier()

        def group(g, carry):
            base = g * G
            pltpu.sync_copy(src_hbm.at[s, pl.ds(base, G)], src_v)
            pltpu.sync_copy(dst_hbm.at[s, pl.ds(base, G)], dst_v)
            for b in range(NBUF - KA):
                pltpu.async_copy(xc_s.at[src_v.at[b]], rows_v.at[b],
                                 gsem.at[b])
            for i in range(G):
                b = i % NBUF
                pltpu.make_async_copy(
                    xc_s.at[src_v.at[i]], rows_v.at[b], gsem.at[b]).wait()
                pltpu.async_copy(rows_v.at[b], agg_s.at[dst_v.at[i]],
                                 ssem.at[b], add=True)
                for r in range(CB // 16):
                    dv = dst_v[i, pl.ds(r * 16, 16)]
                    plsc.addupdate_scatter(dloc_v, [dv], ones16)
                ni = i + NBUF - KA
                if ni < G:
                    nb = ni % NBUF
                    if i >= KA:
                        pltpu.make_async_copy(
                            rows_v.at[nb], agg_s.at[dst_v.at[i - KA]],
                            ssem.at[nb]).wait()
                    pltpu.async_copy(xc_s.at[src_v.at[ni]], rows_v.at[nb],
                                     gsem.at[nb])
            for i in range(G - NBUF, G):
                pltpu.make_async_copy(
                    rows_v.at[i % NBUF], agg_s.at[dst_v.at[i]],
                    ssem.at[i % NBUF]).wait()
            return carry

        lax.fori_loop(0, NG, group, 0)

        pltpu.sync_copy(dloc_v, deg_out.at[c, s])

        plsc.subcore_barrier()

        pltpu.sync_copy(agg_s.at[pl.ds(row0, RPT)],
                        agg_out.at[c, pl.ds(row0, RPT)])

    return agg_kernel(x2cat, src4, dst4, zeros)


def _tc_update(agg2, deg2, x_pad, W, B):
    BR = 512
    grid = (N_PAD // BR,)

    def body(agg_ref, deg_ref, x_ref, w_ref, b_ref, o_ref):
        agg = jnp.concatenate([agg_ref[0], agg_ref[1]], axis=1)
        dg = jnp.sum(deg_ref[...], axis=(0, 1)) * 0.5
        dg = jnp.where(dg == 0.0, 1.0, dg)
        aggn = agg / dg[:, None]
        mm1 = lax.dot_general(aggn, w_ref[...], (((1,), (1,)), ((), ())),
                              preferred_element_type=jnp.float32)
        mm2 = lax.dot_general(x_ref[...], b_ref[...], (((1,), (1,)), ((), ())),
                              preferred_element_type=jnp.float32)
        o_ref[...] = mm1 + mm2

    return pl.pallas_call(
        body,
        grid=grid,
        in_specs=[
            pl.BlockSpec((NC, BR, DH), lambda i: (0, i, 0)),
            pl.BlockSpec((NC, NS, BR), lambda i: (0, 0, i)),
            pl.BlockSpec((BR, D), lambda i: (i, 0)),
            pl.BlockSpec((D, D), lambda i: (0, 0)),
            pl.BlockSpec((D, D), lambda i: (0, 0)),
        ],
        out_specs=pl.BlockSpec((BR, D), lambda i: (i, 0)),
        out_shape=jax.ShapeDtypeStruct((N_PAD, D), jnp.float32),
    )(agg2, deg2, x_pad, W, B)


def kernel(x, edge_index, W, B):
    src = edge_index[0].astype(jnp.int32)
    dst = edge_index[1].astype(jnp.int32)
    pad = E_PAD - N_EDGES
    src = jnp.concatenate([src, jnp.zeros((pad,), jnp.int32)])
    dst = jnp.concatenate([dst, jnp.full((pad,), N_NODES, jnp.int32)])
    src4 = src.reshape(NS, K, CB)
    dst4 = dst.reshape(NS, K, CB)

    x_pad = jnp.zeros((N_PAD, D), jnp.float32).at[:N_NODES].set(x)
    x2cat = x_pad.reshape(N_PAD, 2, DH).swapaxes(0, 1).reshape(2 * N_PAD, DH)
    zeros = jnp.zeros((RPT, DH), jnp.float32)

    agg2, deg2 = _sc_aggregate(x2cat, src4, dst4, zeros)
    out = _tc_update(agg2, deg2, x_pad, W, B)
    return out[:N_NODES]
